# baseline (speedup 1.0000x reference)
import functools

import jax
import jax.numpy as jnp
from jax import lax
from jax.experimental import pallas as pl
from jax.experimental.pallas import tpu as pltpu
from jax.experimental.pallas import tpu_sc as plsc

_K = 8192
_D = 32
_M_TILE = 512
_N_TILES = 8192 // _M_TILE

_NC = 2
_NS = 16
_NW = _NC * _NS
_B_PER_W = 8192 // _NW
_IDX_CHUNK = 128


_K_CHUNK = 2048
_N_CHUNKS = _K // _K_CHUNK


def _argmin_body(x_ref, cbt2_ref, x2_ref, out_ref):
    x = x_ref[...]
    x2 = x2_ref[...]
    acc = jnp.full((_M_TILE,), jnp.inf, jnp.float32)
    aidx = jnp.zeros((_M_TILE,), jnp.int32)
    for c in range(_N_CHUNKS):
        cbt2_c = cbt2_ref[:, c * _K_CHUNK:(c + 1) * _K_CHUNK]
        e2_c = 0.25 * jnp.sum(cbt2_c * cbt2_c, axis=0, keepdims=True)
        xe2 = lax.dot_general(x, cbt2_c, (((1,), (0,)), ((), ())),
                              preferred_element_type=jnp.float32)
        blk = (x2 + e2_c) + xe2
        bmin = jnp.min(blk, axis=1)
        ids = lax.broadcasted_iota(jnp.int32, blk.shape, 1)
        bidx = jnp.min(jnp.where(blk == bmin[:, None], ids, jnp.int32(_K)),
                       axis=1) + jnp.int32(c * _K_CHUNK)
        lt = bmin < acc
        aidx = jnp.where(lt, bidx, aidx)
        acc = jnp.where(lt, bmin.astype(jnp.bfloat16).astype(jnp.float32), acc)
    out_ref[0, 0, :] = aidx


def _nearest_indices(x_flat, cbt2, x2):
    out = pl.pallas_call(
        _argmin_body,
        grid=(_N_TILES,),
        in_specs=[
            pl.BlockSpec((_M_TILE, _D), lambda i: (i, 0)),
            pl.BlockSpec((_D, _K), lambda i: (0, 0)),
            pl.BlockSpec((_M_TILE, 1), lambda i: (i, 0)),
        ],
        out_specs=pl.BlockSpec((1, 1, _M_TILE), lambda i: (i, 0, 0)),
        out_shape=jax.ShapeDtypeStruct((_N_TILES, 1, _M_TILE), jnp.int32),
    )(x_flat, cbt2, x2)
    return out.reshape(-1)


def _sc_gather_body(table_hbm, idx_hbm, out_hbm, idx_v, rows_v, sem):
    wid = lax.axis_index("s") * _NC + lax.axis_index("c")
    base = wid * _B_PER_W
    pltpu.sync_copy(idx_hbm.at[pl.ds(base, _B_PER_W)], idx_v)
    copies = [
        pltpu.async_copy(
            table_hbm.at[idx_v.at[pl.ds(c * _IDX_CHUNK, _IDX_CHUNK)]],
            rows_v.at[pl.ds(c * _IDX_CHUNK, _IDX_CHUNK), :],
            sem,
        )
        for c in range(_B_PER_W // _IDX_CHUNK)
    ]
    for cp in copies:
        cp.wait()
    pltpu.sync_copy(rows_v, out_hbm.at[pl.ds(base, _B_PER_W)])


@functools.cache
def _sc_gather():
    return pl.kernel(
        _sc_gather_body,
        out_type=jax.ShapeDtypeStruct((8192, _D), jnp.float32),
        mesh=plsc.VectorSubcoreMesh(core_axis_name="c", subcore_axis_name="s",
                                    num_cores=_NC, num_subcores=_NS),
        scratch_types=[
            pltpu.VMEM((_B_PER_W,), jnp.int32),
            pltpu.VMEM((_B_PER_W, _D), jnp.float32),
            pltpu.SemaphoreType.DMA,
        ],
        compiler_params=pltpu.CompilerParams(use_tc_tiling_on_sc=False),
    )


def kernel(x, codebook):
    orig_shape = x.shape
    x_flat = x.reshape(-1, codebook.shape[1])
    cbt2 = codebook.T * -2.0
    x2 = jnp.sum(x_flat ** 2, axis=1, keepdims=True)
    indices = _nearest_indices(x_flat, cbt2, x2)
    x_q = _sc_gather()(codebook, indices)
    return (x_q.reshape(orig_shape), indices.reshape(orig_shape[:-1]))


# speedup vs baseline: 1.1011x; 1.1011x over previous
import functools

import jax
import jax.numpy as jnp
from jax import lax
from jax.experimental import pallas as pl
from jax.experimental.pallas import tpu as pltpu
from jax.experimental.pallas import tpu_sc as plsc

_K = 8192
_D = 32
_M_TILE = 512
_N_TILES = 8192 // _M_TILE

_NC = 2
_NS = 16
_NW = _NC * _NS
_B_PER_W = 8192 // _NW
_IDX_CHUNK = 128


_K_CHUNK = 2048
_N_CHUNKS = _K // _K_CHUNK


def _argmin_body(x_ref, cbt2_ref, x2_ref, out_ref):
    x = x_ref[...]
    x2 = x2_ref[...]
    acc = jnp.full((_M_TILE, 1), jnp.inf, jnp.float32)
    win = jnp.zeros((_M_TILE, 1), jnp.int32)
    vwin = jnp.zeros((_M_TILE, 1), jnp.float32)
    blks = []
    for c in range(_N_CHUNKS):
        cbt2_c = cbt2_ref[:, c * _K_CHUNK:(c + 1) * _K_CHUNK]
        e2_c = 0.25 * jnp.sum(cbt2_c * cbt2_c, axis=0, keepdims=True)
        xe2 = lax.dot_general(x, cbt2_c, (((1,), (0,)), ((), ())),
                              preferred_element_type=jnp.float32)
        blk = (x2 + e2_c) + xe2
        blks.append(blk)
        bmin = jnp.min(blk, axis=1, keepdims=True)
        lt = bmin < acc
        win = jnp.where(lt, jnp.int32(c), win)
        vwin = jnp.where(lt, bmin, vwin)
        acc = jnp.where(lt, bmin.astype(jnp.bfloat16).astype(jnp.float32), acc)
    t01 = jnp.where(win == 0, blks[0], blks[1])
    t23 = jnp.where(win == 2, blks[2], blks[3])
    blk_w = jnp.where(win < 2, t01, t23)
    ids = lax.broadcasted_iota(jnp.int32, blk_w.shape, 1)
    bidx = jnp.min(jnp.where(blk_w == vwin, ids, jnp.int32(_K)), axis=1)
    out_ref[0, 0, :] = bidx + win[:, 0] * jnp.int32(_K_CHUNK)


def _nearest_indices(x_flat, cbt2, x2):
    out = pl.pallas_call(
        _argmin_body,
        grid=(_N_TILES,),
        in_specs=[
            pl.BlockSpec((_M_TILE, _D), lambda i: (i, 0)),
            pl.BlockSpec((_D, _K), lambda i: (0, 0)),
            pl.BlockSpec((_M_TILE, 1), lambda i: (i, 0)),
        ],
        out_specs=pl.BlockSpec((1, 1, _M_TILE), lambda i: (i, 0, 0)),
        out_shape=jax.ShapeDtypeStruct((_N_TILES, 1, _M_TILE), jnp.int32),
    )(x_flat, cbt2, x2)
    return out.reshape(-1)


def _sc_gather_body(table_hbm, idx_hbm, out_hbm, idx_v, rows_v, sem):
    wid = lax.axis_index("s") * _NC + lax.axis_index("c")
    base = wid * _B_PER_W
    pltpu.sync_copy(idx_hbm.at[pl.ds(base, _B_PER_W)], idx_v)
    copies = [
        pltpu.async_copy(
            table_hbm.at[idx_v.at[pl.ds(c * _IDX_CHUNK, _IDX_CHUNK)]],
            rows_v.at[pl.ds(c * _IDX_CHUNK, _IDX_CHUNK), :],
            sem,
        )
        for c in range(_B_PER_W // _IDX_CHUNK)
    ]
    for cp in copies:
        cp.wait()
    pltpu.sync_copy(rows_v, out_hbm.at[pl.ds(base, _B_PER_W)])


@functools.cache
def _sc_gather():
    return pl.kernel(
        _sc_gather_body,
        out_type=jax.ShapeDtypeStruct((8192, _D), jnp.float32),
        mesh=plsc.VectorSubcoreMesh(core_axis_name="c", subcore_axis_name="s",
                                    num_cores=_NC, num_subcores=_NS),
        scratch_types=[
            pltpu.VMEM((_B_PER_W,), jnp.int32),
            pltpu.VMEM((_B_PER_W, _D), jnp.float32),
            pltpu.SemaphoreType.DMA,
        ],
        compiler_params=pltpu.CompilerParams(use_tc_tiling_on_sc=False),
    )


def kernel(x, codebook):
    orig_shape = x.shape
    x_flat = x.reshape(-1, codebook.shape[1])
    cbt2 = codebook.T * -2.0
    x2 = jnp.sum(x_flat ** 2, axis=1, keepdims=True)
    indices = _nearest_indices(x_flat, cbt2, x2)
    x_q = _sc_gather()(codebook, indices)
    return (x_q.reshape(orig_shape), indices.reshape(orig_shape[:-1]))


# speedup vs baseline: 1.1163x; 1.0138x over previous
import functools

import jax
import jax.numpy as jnp
from jax import lax
from jax.experimental import pallas as pl
from jax.experimental.pallas import tpu as pltpu
from jax.experimental.pallas import tpu_sc as plsc

_K = 8192
_D = 32
_M_TILE = 1024
_N_TILES = 8192 // _M_TILE

_NC = 2
_NS = 16
_NW = _NC * _NS
_B_PER_W = 8192 // _NW
_IDX_CHUNK = 128


_K_CHUNK = 2048
_N_CHUNKS = _K // _K_CHUNK


def _argmin_body(x_ref, cbt2_ref, x2_ref, out_ref):
    x = x_ref[...]
    x2 = x2_ref[...]
    acc = jnp.full((_M_TILE, 1), jnp.inf, jnp.float32)
    win = jnp.zeros((_M_TILE, 1), jnp.int32)
    vwin = jnp.zeros((_M_TILE, 1), jnp.float32)
    blks = []
    for c in range(_N_CHUNKS):
        cbt2_c = cbt2_ref[:, c * _K_CHUNK:(c + 1) * _K_CHUNK]
        e2_c = 0.25 * jnp.sum(cbt2_c * cbt2_c, axis=0, keepdims=True)
        xe2 = lax.dot_general(x, cbt2_c, (((1,), (0,)), ((), ())),
                              preferred_element_type=jnp.float32)
        blk = (x2 + e2_c) + xe2
        blks.append(blk)
        bmin = jnp.min(blk, axis=1, keepdims=True)
        lt = bmin < acc
        win = jnp.where(lt, jnp.int32(c), win)
        vwin = jnp.where(lt, bmin, vwin)
        acc = jnp.where(lt, bmin.astype(jnp.bfloat16).astype(jnp.float32), acc)
    t01 = jnp.where(win == 0, blks[0], blks[1])
    t23 = jnp.where(win == 2, blks[2], blks[3])
    blk_w = jnp.where(win < 2, t01, t23)
    ids = lax.broadcasted_iota(jnp.int32, blk_w.shape, 1)
    bidx = jnp.min(jnp.where(blk_w == vwin, ids, jnp.int32(_K)), axis=1)
    out_ref[0, 0, :] = bidx + win[:, 0] * jnp.int32(_K_CHUNK)


def _nearest_indices(x_flat, cbt2, x2):
    out = pl.pallas_call(
        _argmin_body,
        grid=(_N_TILES,),
        in_specs=[
            pl.BlockSpec((_M_TILE, _D), lambda i: (i, 0)),
            pl.BlockSpec((_D, _K), lambda i: (0, 0)),
            pl.BlockSpec((_M_TILE, 1), lambda i: (i, 0)),
        ],
        out_specs=pl.BlockSpec((1, 1, _M_TILE), lambda i: (i, 0, 0)),
        out_shape=jax.ShapeDtypeStruct((_N_TILES, 1, _M_TILE), jnp.int32),
    )(x_flat, cbt2, x2)
    return out.reshape(-1)


def _sc_gather_body(table_hbm, idx_hbm, out_hbm, idx_v, rows_v, sem):
    wid = lax.axis_index("s") * _NC + lax.axis_index("c")
    base = wid * _B_PER_W
    pltpu.sync_copy(idx_hbm.at[pl.ds(base, _B_PER_W)], idx_v)
    copies = [
        pltpu.async_copy(
            table_hbm.at[idx_v.at[pl.ds(c * _IDX_CHUNK, _IDX_CHUNK)]],
            rows_v.at[pl.ds(c * _IDX_CHUNK, _IDX_CHUNK), :],
            sem,
        )
        for c in range(_B_PER_W // _IDX_CHUNK)
    ]
    for cp in copies:
        cp.wait()
    pltpu.sync_copy(rows_v, out_hbm.at[pl.ds(base, _B_PER_W)])


@functools.cache
def _sc_gather():
    return pl.kernel(
        _sc_gather_body,
        out_type=jax.ShapeDtypeStruct((8192, _D), jnp.float32),
        mesh=plsc.VectorSubcoreMesh(core_axis_name="c", subcore_axis_name="s",
                                    num_cores=_NC, num_subcores=_NS),
        scratch_types=[
            pltpu.VMEM((_B_PER_W,), jnp.int32),
            pltpu.VMEM((_B_PER_W, _D), jnp.float32),
            pltpu.SemaphoreType.DMA,
        ],
        compiler_params=pltpu.CompilerParams(use_tc_tiling_on_sc=False),
    )


def kernel(x, codebook):
    orig_shape = x.shape
    x_flat = x.reshape(-1, codebook.shape[1])
    cbt2 = codebook.T * -2.0
    x2 = jnp.sum(x_flat ** 2, axis=1, keepdims=True)
    indices = _nearest_indices(x_flat, cbt2, x2)
    x_q = _sc_gather()(codebook, indices)
    return (x_q.reshape(orig_shape), indices.reshape(orig_shape[:-1]))
